# baseline (speedup 1.0000x reference)
import functools
import math

import jax
import jax.numpy as jnp
from jax import lax
from jax.experimental import pallas as pl
from jax.experimental.pallas import tpu as pltpu
from jax.experimental.pallas import tpu_sc as plsc

jax.config.update("jax_enable_x64", True)

V = 32000
N = 2048
SMOOTHING = 0.1
CONF = 1.0 - SMOOTHING
SVAL = SMOOTHING / (V - 2)
ROW_TLOGT = (V - 2) * SVAL * math.log(SVAL) + CONF * math.log(CONF)

NTC = 1024
ROW_BLK = 256
COL_BLK = 6400

_NW = 32
_BPW = N // _NW
_RPT = (N - NTC) // _NW


def _tc_body(w_ref, x_ref, o_ref):
    @pl.when((pl.program_id(0) == 0) & (pl.program_id(1) == 0))
    def _init():
        o_ref[...] = jnp.zeros_like(o_ref)

    o_ref[...] += jnp.sum(x_ref[...] * w_ref[...])


def _masked_total_sum(output, w):
    return pl.pallas_call(
        _tc_body,
        grid=(NTC // ROW_BLK, V // COL_BLK),
        in_specs=[
            pl.BlockSpec((ROW_BLK, 1), lambda i, j: (i, jnp.int32(0))),
            pl.BlockSpec((ROW_BLK, COL_BLK), lambda i, j: (i, j)),
        ],
        out_specs=pl.BlockSpec((1, 1),
                               lambda i, j: (jnp.int32(0), jnp.int32(0))),
        out_shape=jax.ShapeDtypeStruct((1, 1), jnp.float32),
        compiler_params=pltpu.CompilerParams(
            dimension_semantics=("arbitrary", "arbitrary")),
    )(w, output)


def _sc_partials(out_flat, tgt32):
    mesh = plsc.VectorSubcoreMesh(core_axis_name="c", subcore_axis_name="s")

    @functools.partial(
        pl.kernel,
        mesh=mesh,
        out_type=jax.ShapeDtypeStruct((_NW, 4, 16), jnp.float32),
        scratch_types=[
            pltpu.VMEM((_BPW,), jnp.int32),
            pltpu.VMEM((_RPT,), jnp.int32),
            pltpu.VMEM((2 * _BPW,), jnp.int32),
            pltpu.VMEM((2 * _BPW,), jnp.float32),
            pltpu.VMEM((V,), jnp.float32),
            pltpu.VMEM((V,), jnp.float32),
            pltpu.VMEM((4, 16), jnp.float32),
            pltpu.SemaphoreType.DMA,
            pltpu.SemaphoreType.DMA,
            pltpu.SemaphoreType.DMA,
        ],
    )
    def sc_k(x_hbm, t_hbm, o_hbm, t_v, td_v, idx_v, g_v, b0, b1, acc_v,
             sem, sd0, sd1):
        wid = lax.axis_index("s") * 2 + lax.axis_index("c")
        zeros = jnp.zeros((16,), jnp.float32)
        ones = jnp.ones((16,), jnp.float32)
        izeros = jnp.zeros((16,), jnp.int32)

        base = wid * _BPW
        pltpu.sync_copy(t_hbm.at[pl.ds(base, _BPW)], t_v)
        for j in range(_BPW // 16):
            t16 = t_v[pl.ds(j * 16, 16)]
            row16 = base + j * 16 + lax.iota(jnp.int32, 16)
            idx_v[pl.ds(j * 16, 16)] = row16 * V + t16
            idx_v[pl.ds(_BPW + j * 16, 16)] = row16 * V
        gather_cp = pltpu.async_copy(x_hbm.at[idx_v], g_v, sem)

        rbase = NTC + wid * _RPT
        pltpu.sync_copy(t_hbm.at[pl.ds(rbase, _RPT)], td_v)
        bufs = (b0, b1)
        sems = (sd0, sd1)
        handles = {}
        handles[0] = pltpu.async_copy(
            x_hbm.at[pl.ds(rbase * V, V)], b0, sd0)
        accd = zeros
        for r in range(_RPT):
            if r + 1 < _RPT:
                handles[r + 1] = pltpu.async_copy(
                    x_hbm.at[pl.ds((rbase + r + 1) * V, V)],
                    bufs[(r + 1) % 2], sems[(r + 1) % 2])
            handles[r].wait()
            buf = bufs[r % 2]

            def inner(j, a, buf=buf):
                o = j * jnp.int32(128)
                for k in range(8):
                    a = a + buf[pl.ds(o + jnp.int32(k * 16), 16)]
                return a

            rowacc = lax.fori_loop(jnp.int32(0), jnp.int32(V // 128),
                                   inner, zeros)
            t16 = td_v[pl.ds((r // 16) * 16, 16)]
            w16 = jnp.where(t16 != izeros, ones, zeros)
            wsplat = lax.gather(
                w16, jnp.full((16, 1), r % 16, jnp.int32),
                lax.GatherDimensionNumbers(offset_dims=(),
                                           collapsed_slice_dims=(0,),
                                           start_index_map=(0,)),
                (1,), mode=lax.GatherScatterMode.PROMISE_IN_BOUNDS)
            accd = accd + rowacc * wsplat

        gather_cp.wait()
        accg = zeros
        accz = zeros
        accn = zeros
        for j in range(_BPW // 16):
            t16 = t_v[pl.ds(j * 16, 16)]
            m16 = t16 != izeros
            accg = accg + jnp.where(m16, g_v[pl.ds(j * 16, 16)], zeros)
            accz = accz + jnp.where(m16, g_v[pl.ds(_BPW + j * 16, 16)], zeros)
            accn = accn + jnp.where(m16, ones, zeros)
        acc_v[0] = accg
        acc_v[1] = accz
        acc_v[2] = accn
        acc_v[3] = accd
        pltpu.sync_copy(acc_v, o_hbm.at[wid])

    return sc_k(out_flat, tgt32)


def kernel(output, target):
    tgt32 = target.astype(jnp.int32)
    w = (tgt32 != 0).astype(jnp.float32)[:, None]
    t_tc = _masked_total_sum(output, w)[0, 0]
    parts = _sc_partials(output.reshape(-1), tgt32)
    g64 = jnp.sum(parts[:, 0, :]).astype(jnp.float64)
    z64 = jnp.sum(parts[:, 1, :]).astype(jnp.float64)
    n64 = jnp.sum(parts[:, 2, :]).astype(jnp.float64)
    t64 = (t_tc + jnp.sum(parts[:, 3, :])).astype(jnp.float64)
    return n64 * ROW_TLOGT - SVAL * t64 + SVAL * z64 + (SVAL - CONF) * g64


# speedup vs baseline: 2.4044x; 2.4044x over previous
import functools
import math

import numpy as np

import jax
import jax.numpy as jnp
from jax import lax
from jax.experimental import pallas as pl
from jax.experimental.pallas import tpu as pltpu
from jax.experimental.pallas import tpu_sc as plsc

jax.config.update("jax_enable_x64", True)

V = 32000
N = 2048
SMOOTHING = 0.1
CONF = 1.0 - SMOOTHING
SVAL = SMOOTHING / (V - 2)
ROW_TLOGT = (V - 2) * SVAL * math.log(SVAL) + CONF * math.log(CONF)

NTC = 1536
ROW_BLK = 256
COL_BLK = 6400

_NW = 32
_RPT = (N - NTC) // _NW
_CCH = 6400
_NCH = V // _CCH


def _tc_body(w_ref, t_ref, x_ref, tsum_ref, g_ref, z_ref, n_ref):
    i = pl.program_id(0)
    j = pl.program_id(1)

    @pl.when((i == 0) & (j == 0))
    def _init():
        tsum_ref[...] = jnp.zeros_like(tsum_ref)
        g_ref[...] = jnp.zeros_like(g_ref)
        z_ref[...] = jnp.zeros_like(z_ref)
        n_ref[...] = jnp.zeros_like(n_ref)

    xw = x_ref[...] * w_ref[...]
    tsum_ref[...] += jnp.sum(xw)
    cols = jax.lax.broadcasted_iota(jnp.int32, (ROW_BLK, COL_BLK), 1) \
        + j * COL_BLK
    g_ref[...] += jnp.sum(jnp.where(cols == t_ref[...], xw, 0.0))

    @pl.when(j == 0)
    def _col0():
        z_ref[...] += jnp.sum(xw[:, 0:1])
        n_ref[...] += jnp.sum(w_ref[...])


def _tc_sums(output, w, t2d):
    zero2 = lambda i, j: (jnp.int32(0), jnp.int32(0))
    one = jax.ShapeDtypeStruct((1, 1), jnp.float32)
    return pl.pallas_call(
        _tc_body,
        grid=(NTC // ROW_BLK, V // COL_BLK),
        in_specs=[
            pl.BlockSpec((ROW_BLK, 1), lambda i, j: (i, jnp.int32(0))),
            pl.BlockSpec((ROW_BLK, 1), lambda i, j: (i, jnp.int32(0))),
            pl.BlockSpec((ROW_BLK, COL_BLK), lambda i, j: (i, j)),
        ],
        out_specs=[pl.BlockSpec((1, 1), zero2)] * 4,
        out_shape=[one, one, one, one],
        compiler_params=pltpu.CompilerParams(
            dimension_semantics=("arbitrary", "arbitrary")),
    )(w, t2d, output)


def _lane_splat(vec16, lane):
    return lax.gather(
        vec16, jnp.full((16, 1), lane, jnp.int32),
        lax.GatherDimensionNumbers(offset_dims=(),
                                   collapsed_slice_dims=(0,),
                                   start_index_map=(0,)),
        (1,), mode=lax.GatherScatterMode.PROMISE_IN_BOUNDS)


def _sc_partials(x2d, tgtf, wf):
    mesh = plsc.VectorSubcoreMesh(core_axis_name="c", subcore_axis_name="s")

    @functools.partial(
        pl.kernel,
        mesh=mesh,
        out_type=jax.ShapeDtypeStruct((_NW, 4, 16), jnp.float32),
        scratch_types=[
            pltpu.VMEM((_RPT,), jnp.float32),
            pltpu.VMEM((_RPT,), jnp.float32),
            pltpu.VMEM((8, _CCH), jnp.float32),
            pltpu.VMEM((8, _CCH), jnp.float32),
            pltpu.VMEM((4, 16), jnp.float32),
            pltpu.SemaphoreType.DMA,
            pltpu.SemaphoreType.DMA,
        ],
    )
    def sc_k(x_hbm, tf_hbm, wf_hbm, o_hbm, tf_v, wf_v, b0, b1, acc_v,
             sd0, sd1):
        wid = lax.axis_index("s") * 2 + lax.axis_index("c")
        rbase = NTC + wid * _RPT
        zeros = jnp.zeros((16,), jnp.float32)
        ones = jnp.ones((16,), jnp.float32)
        iota_f = lax.iota(jnp.int32, 16).astype(jnp.float32)
        e0 = jnp.maximum(ones - iota_f * iota_f, 0.0)
        pltpu.sync_copy(tf_hbm.at[pl.ds(rbase, _RPT)], tf_v)
        pltpu.sync_copy(wf_hbm.at[pl.ds(rbase, _RPT)], wf_v)

        accn = zeros
        for k in range(_RPT // 16):
            accn = accn + wf_v[pl.ds(k * 16, 16)]

        seq = [(g, c) for g in range(_RPT // 8) for c in range(_NCH)]
        bufs = (b0, b1)
        sems = (sd0, sd1)

        def start(k):
            g, c = seq[k]
            return pltpu.async_copy(
                x_hbm.at[pl.ds(rbase + 8 * g, 8), pl.ds(c * _CCH, _CCH)],
                bufs[k % 2], sems[k % 2])

        handles = {0: start(0)}
        acct = zeros
        accg = zeros
        accz = zeros
        for k, (g, c) in enumerate(seq):
            if k + 1 < len(seq):
                handles[k + 1] = start(k + 1)
            handles[k].wait()
            buf = bufs[k % 2]
            for l in range(8):
                rlane = g * 8 + l
                tch = tf_v[pl.ds((rlane // 16) * 16, 16)]
                wch = wf_v[pl.ds((rlane // 16) * 16, 16)]
                tspl = _lane_splat(tch, rlane % 16)
                wspl = _lane_splat(wch, rlane % 16)
                cvec0 = iota_f + jnp.float32(c * _CCH)

                def inner(j, carry, buf=buf, l=l, tspl=tspl):
                    racc, gacc, cvec = carry
                    o = j * jnp.int32(64)
                    for u in range(4):
                        x16 = buf[l, pl.ds(o + jnp.int32(u * 16), 16)]
                        racc = racc + x16
                        d = cvec - tspl
                        ind = jnp.maximum(ones - d * d, 0.0)
                        gacc = gacc + x16 * ind
                        cvec = cvec + jnp.float32(16.0)
                    return racc, gacc, cvec

                racc, gacc, _ = lax.fori_loop(
                    jnp.int32(0), jnp.int32(_CCH // 64), inner,
                    (zeros, zeros, cvec0))
                acct = acct + racc * wspl
                accg = accg + gacc * wspl
                if c == 0:
                    accz = accz + buf[l, pl.ds(0, 16)] * e0 * wspl
        acc_v[0] = accg
        acc_v[1] = accz
        acc_v[2] = accn
        acc_v[3] = acct
        pltpu.sync_copy(acc_v, o_hbm.at[wid])

    return sc_k(x2d, tgtf, wf)


def kernel(output, target):
    tgt32 = target.astype(jnp.int32)
    wcol = (tgt32 != 0).astype(jnp.float32)[:, None]
    tcol = tgt32[:, None]
    tc_t, tc_g, tc_z, tc_n = _tc_sums(output, wcol, tcol)
    parts = _sc_partials(output, tgt32.astype(jnp.float32), wcol[:, 0])
    g64 = (tc_g[0, 0] + jnp.sum(parts[:, 0, :])).astype(jnp.float64)
    z64 = (tc_z[0, 0] + jnp.sum(parts[:, 1, :])).astype(jnp.float64)
    n64 = (tc_n[0, 0] + jnp.sum(parts[:, 2, :])).astype(jnp.float64)
    t64 = (tc_t[0, 0] + jnp.sum(parts[:, 3, :])).astype(jnp.float64)
    return n64 * ROW_TLOGT - SVAL * t64 + SVAL * z64 + (SVAL - CONF) * g64


# speedup vs baseline: 2.5009x; 1.0401x over previous
import functools
import math

import numpy as np

import jax
import jax.numpy as jnp
from jax import lax
from jax.experimental import pallas as pl
from jax.experimental.pallas import tpu as pltpu
from jax.experimental.pallas import tpu_sc as plsc

jax.config.update("jax_enable_x64", True)

V = 32000
N = 2048
SMOOTHING = 0.1
CONF = 1.0 - SMOOTHING
SVAL = SMOOTHING / (V - 2)
ROW_TLOGT = (V - 2) * SVAL * math.log(SVAL) + CONF * math.log(CONF)

NTC = 1280
ROW_BLK = 256
COL_BLK = 6400

_NW = 32
_RPT = (N - NTC) // _NW
_CCH = 6400
_NCH = V // _CCH
_RPAD = ((_RPT + 15) // 16) * 16


def _tc_body(w_ref, t_ref, x_ref, tsum_ref, g_ref, z_ref, n_ref):
    i = pl.program_id(0)
    j = pl.program_id(1)

    @pl.when((i == 0) & (j == 0))
    def _init():
        tsum_ref[...] = jnp.zeros_like(tsum_ref)
        g_ref[...] = jnp.zeros_like(g_ref)
        z_ref[...] = jnp.zeros_like(z_ref)
        n_ref[...] = jnp.zeros_like(n_ref)

    xw = x_ref[...] * w_ref[...]
    tsum_ref[...] += jnp.sum(xw)
    cols = jax.lax.broadcasted_iota(jnp.int32, (ROW_BLK, COL_BLK), 1) \
        + j * COL_BLK
    g_ref[...] += jnp.sum(jnp.where(cols == t_ref[...], xw, 0.0))

    @pl.when(j == 0)
    def _col0():
        z_ref[...] += jnp.sum(xw[:, 0:1])
        n_ref[...] += jnp.sum(w_ref[...])


def _tc_sums(output, w, t2d):
    zero2 = lambda i, j: (jnp.int32(0), jnp.int32(0))
    one = jax.ShapeDtypeStruct((1, 1), jnp.float32)
    return pl.pallas_call(
        _tc_body,
        grid=(NTC // ROW_BLK, V // COL_BLK),
        in_specs=[
            pl.BlockSpec((ROW_BLK, 1), lambda i, j: (i, jnp.int32(0))),
            pl.BlockSpec((ROW_BLK, 1), lambda i, j: (i, jnp.int32(0))),
            pl.BlockSpec((ROW_BLK, COL_BLK), lambda i, j: (i, j)),
        ],
        out_specs=[pl.BlockSpec((1, 1), zero2)] * 4,
        out_shape=[one, one, one, one],
        compiler_params=pltpu.CompilerParams(
            dimension_semantics=("arbitrary", "arbitrary")),
    )(w, t2d, output)


def _lane_splat(vec16, lane):
    return lax.gather(
        vec16, jnp.full((16, 1), lane, jnp.int32),
        lax.GatherDimensionNumbers(offset_dims=(),
                                   collapsed_slice_dims=(0,),
                                   start_index_map=(0,)),
        (1,), mode=lax.GatherScatterMode.PROMISE_IN_BOUNDS)


def _sc_partials(x2d, tgtf, wf):
    mesh = plsc.VectorSubcoreMesh(core_axis_name="c", subcore_axis_name="s")

    @functools.partial(
        pl.kernel,
        mesh=mesh,
        out_type=jax.ShapeDtypeStruct((_NW, 4, 16), jnp.float32),
        scratch_types=[
            pltpu.VMEM((_RPAD,), jnp.float32),
            pltpu.VMEM((_RPAD,), jnp.float32),
            pltpu.VMEM((8, _CCH), jnp.float32),
            pltpu.VMEM((8, _CCH), jnp.float32),
            pltpu.VMEM((4, 16), jnp.float32),
            pltpu.SemaphoreType.DMA,
            pltpu.SemaphoreType.DMA,
        ],
    )
    def sc_k(x_hbm, tf_hbm, wf_hbm, o_hbm, tf_v, wf_v, b0, b1, acc_v,
             sd0, sd1):
        wid = lax.axis_index("s") * 2 + lax.axis_index("c")
        rbase = NTC + wid * _RPT
        zeros = jnp.zeros((16,), jnp.float32)
        ones = jnp.ones((16,), jnp.float32)
        iota_f = lax.iota(jnp.int32, 16).astype(jnp.float32)
        e0 = jnp.maximum(ones - iota_f * iota_f, 0.0)
        for k in range(_RPAD // 16):
            wf_v[pl.ds(k * 16, 16)] = zeros
        pltpu.sync_copy(tf_hbm.at[pl.ds(rbase, _RPT)], tf_v.at[pl.ds(0, _RPT)])
        pltpu.sync_copy(wf_hbm.at[pl.ds(rbase, _RPT)], wf_v.at[pl.ds(0, _RPT)])

        accn = zeros
        for k in range(_RPAD // 16):
            accn = accn + wf_v[pl.ds(k * 16, 16)]

        seq = [(g, c) for g in range(_RPT // 8) for c in range(_NCH)]
        bufs = (b0, b1)
        sems = (sd0, sd1)

        def start(k):
            g, c = seq[k]
            return pltpu.async_copy(
                x_hbm.at[pl.ds(rbase + 8 * g, 8), pl.ds(c * _CCH, _CCH)],
                bufs[k % 2], sems[k % 2])

        handles = {0: start(0)}
        acct = zeros
        accg = zeros
        accz = zeros
        for k, (g, c) in enumerate(seq):
            if k + 1 < len(seq):
                handles[k + 1] = start(k + 1)
            handles[k].wait()
            buf = bufs[k % 2]
            for l in range(8):
                rlane = g * 8 + l
                tch = tf_v[pl.ds((rlane // 16) * 16, 16)]
                wch = wf_v[pl.ds((rlane // 16) * 16, 16)]
                tspl = _lane_splat(tch, rlane % 16)
                wspl = _lane_splat(wch, rlane % 16)
                cvec0 = iota_f + jnp.float32(c * _CCH)

                def inner(j, carry, buf=buf, l=l, tspl=tspl):
                    racc, gacc, cvec = carry
                    o = j * jnp.int32(64)
                    for u in range(4):
                        x16 = buf[l, pl.ds(o + jnp.int32(u * 16), 16)]
                        racc = racc + x16
                        gacc = gacc + jnp.where(cvec == tspl, x16, zeros)
                        cvec = cvec + jnp.float32(16.0)
                    return racc, gacc, cvec

                racc, gacc, _ = lax.fori_loop(
                    jnp.int32(0), jnp.int32(_CCH // 64), inner,
                    (zeros, zeros, cvec0))
                acct = acct + racc * wspl
                accg = accg + gacc * wspl
                if c == 0:
                    accz = accz + buf[l, pl.ds(0, 16)] * e0 * wspl
        acc_v[0] = accg
        acc_v[1] = accz
        acc_v[2] = accn
        acc_v[3] = acct
        pltpu.sync_copy(acc_v, o_hbm.at[wid])

    return sc_k(x2d, tgtf, wf)


def kernel(output, target):
    tgt32 = target.astype(jnp.int32)
    wcol = (tgt32 != 0).astype(jnp.float32)[:, None]
    tcol = tgt32[:, None]
    tc_t, tc_g, tc_z, tc_n = _tc_sums(output, wcol, tcol)
    parts = _sc_partials(output, tgt32.astype(jnp.float32), wcol[:, 0])
    g64 = (tc_g[0, 0] + jnp.sum(parts[:, 0, :])).astype(jnp.float64)
    z64 = (tc_z[0, 0] + jnp.sum(parts[:, 1, :])).astype(jnp.float64)
    n64 = (tc_n[0, 0] + jnp.sum(parts[:, 2, :])).astype(jnp.float64)
    t64 = (tc_t[0, 0] + jnp.sum(parts[:, 3, :])).astype(jnp.float64)
    return n64 * ROW_TLOGT - SVAL * t64 + SVAL * z64 + (SVAL - CONF) * g64


# speedup vs baseline: 2.5346x; 1.0135x over previous
import functools
import math

import numpy as np

import jax
import jax.numpy as jnp
from jax import lax
from jax.experimental import pallas as pl
from jax.experimental.pallas import tpu as pltpu
from jax.experimental.pallas import tpu_sc as plsc

jax.config.update("jax_enable_x64", True)

V = 32000
N = 2048
SMOOTHING = 0.1
CONF = 1.0 - SMOOTHING
SVAL = SMOOTHING / (V - 2)
ROW_TLOGT = (V - 2) * SVAL * math.log(SVAL) + CONF * math.log(CONF)

NTC = 1280
ROW_BLK = 256
COL_BLK = 6400

_NW = 32
_RPT = (N - NTC) // _NW
_CCH = 6400
_NCH = V // _CCH
_RPAD = ((_RPT + 15) // 16) * 16


def _tc_body(t_ref, x_ref, tsum_ref, g_ref, z_ref, n_ref):
    i = pl.program_id(0)
    j = pl.program_id(1)

    @pl.when((i == 0) & (j == 0))
    def _init():
        tsum_ref[...] = jnp.zeros_like(tsum_ref)
        g_ref[...] = jnp.zeros_like(g_ref)
        z_ref[...] = jnp.zeros_like(z_ref)
        n_ref[...] = jnp.zeros_like(n_ref)

    w = (t_ref[...] != 0).astype(jnp.float32)
    xw = x_ref[...] * w
    tsum_ref[...] += jnp.sum(xw)
    cols = jax.lax.broadcasted_iota(jnp.int32, (ROW_BLK, COL_BLK), 1) \
        + j * COL_BLK
    g_ref[...] += jnp.sum(jnp.where(cols == t_ref[...], xw, 0.0))

    @pl.when(j == 0)
    def _col0():
        z_ref[...] += jnp.sum(xw[:, 0:1])
        n_ref[...] += jnp.sum(w)


def _tc_sums(output, t2d):
    zero2 = lambda i, j: (jnp.int32(0), jnp.int32(0))
    one = jax.ShapeDtypeStruct((1, 1), jnp.float32)
    return pl.pallas_call(
        _tc_body,
        grid=(NTC // ROW_BLK, V // COL_BLK),
        in_specs=[
            pl.BlockSpec((ROW_BLK, 1), lambda i, j: (i, jnp.int32(0))),
            pl.BlockSpec((ROW_BLK, COL_BLK), lambda i, j: (i, j)),
        ],
        out_specs=[pl.BlockSpec((1, 1), zero2)] * 4,
        out_shape=[one, one, one, one],
        compiler_params=pltpu.CompilerParams(
            dimension_semantics=("arbitrary", "arbitrary")),
    )(t2d, output)


def _lane_splat(vec16, lane):
    return lax.gather(
        vec16, jnp.full((16, 1), lane, jnp.int32),
        lax.GatherDimensionNumbers(offset_dims=(),
                                   collapsed_slice_dims=(0,),
                                   start_index_map=(0,)),
        (1,), mode=lax.GatherScatterMode.PROMISE_IN_BOUNDS)


def _sc_partials(x2d, tgtf):
    mesh = plsc.VectorSubcoreMesh(core_axis_name="c", subcore_axis_name="s")

    @functools.partial(
        pl.kernel,
        mesh=mesh,
        out_type=jax.ShapeDtypeStruct((_NW, 4, 16), jnp.float32),
        scratch_types=[
            pltpu.VMEM((_RPAD,), jnp.float32),
            pltpu.VMEM((_RPAD,), jnp.float32),
            pltpu.VMEM((8, _CCH), jnp.float32),
            pltpu.VMEM((8, _CCH), jnp.float32),
            pltpu.VMEM((4, 16), jnp.float32),
            pltpu.SemaphoreType.DMA,
            pltpu.SemaphoreType.DMA,
        ],
    )
    def sc_k(x_hbm, tf_hbm, o_hbm, tf_v, wf_v, b0, b1, acc_v,
             sd0, sd1):
        wid = lax.axis_index("s") * 2 + lax.axis_index("c")
        rbase = NTC + wid * _RPT
        zeros = jnp.zeros((16,), jnp.float32)
        ones = jnp.ones((16,), jnp.float32)
        iota_f = lax.iota(jnp.int32, 16).astype(jnp.float32)
        e0 = jnp.maximum(ones - iota_f * iota_f, 0.0)
        for k in range(_RPAD // 16):
            tf_v[pl.ds(k * 16, 16)] = zeros
        pltpu.sync_copy(tf_hbm.at[pl.ds(rbase, _RPT)], tf_v.at[pl.ds(0, _RPT)])

        accn = zeros
        for k in range(_RPAD // 16):
            t16 = tf_v[pl.ds(k * 16, 16)]
            w16 = jnp.where(t16 != zeros, ones, zeros)
            if (k + 1) * 16 > _RPT:
                lanes = lax.iota(jnp.int32, 16)
                w16 = jnp.where(lanes < (_RPT - k * 16), w16, zeros)
            wf_v[pl.ds(k * 16, 16)] = w16
            accn = accn + w16

        seq = [(g, c) for g in range(_RPT // 8) for c in range(_NCH)]
        bufs = (b0, b1)
        sems = (sd0, sd1)

        def start(k):
            g, c = seq[k]
            return pltpu.async_copy(
                x_hbm.at[pl.ds(rbase + 8 * g, 8), pl.ds(c * _CCH, _CCH)],
                bufs[k % 2], sems[k % 2])

        handles = {0: start(0)}
        acct = zeros
        accg = zeros
        accz = zeros
        for k, (g, c) in enumerate(seq):
            if k + 1 < len(seq):
                handles[k + 1] = start(k + 1)
            handles[k].wait()
            buf = bufs[k % 2]
            for l in range(8):
                rlane = g * 8 + l
                tch = tf_v[pl.ds((rlane // 16) * 16, 16)]
                wch = wf_v[pl.ds((rlane // 16) * 16, 16)]
                tspl = _lane_splat(tch, rlane % 16)
                wspl = _lane_splat(wch, rlane % 16)
                cvec0 = iota_f + jnp.float32(c * _CCH)

                def inner(j, carry, buf=buf, l=l, tspl=tspl):
                    racc, gacc, cvec = carry
                    o = j * jnp.int32(64)
                    for u in range(4):
                        x16 = buf[l, pl.ds(o + jnp.int32(u * 16), 16)]
                        racc = racc + x16
                        gacc = gacc + jnp.where(cvec == tspl, x16, zeros)
                        cvec = cvec + jnp.float32(16.0)
                    return racc, gacc, cvec

                racc, gacc, _ = lax.fori_loop(
                    jnp.int32(0), jnp.int32(_CCH // 64), inner,
                    (zeros, zeros, cvec0))
                acct = acct + racc * wspl
                accg = accg + gacc * wspl
                if c == 0:
                    accz = accz + buf[l, pl.ds(0, 16)] * e0 * wspl
        acc_v[0] = accg
        acc_v[1] = accz
        acc_v[2] = accn
        acc_v[3] = acct
        pltpu.sync_copy(acc_v, o_hbm.at[wid])

    return sc_k(x2d, tgtf)


def kernel(output, target):
    tgt32 = target.astype(jnp.int32)
    tcol = tgt32[:, None]
    tc_t, tc_g, tc_z, tc_n = _tc_sums(output, tcol)
    parts = _sc_partials(output, tgt32.astype(jnp.float32))
    g64 = (tc_g[0, 0] + jnp.sum(parts[:, 0, :])).astype(jnp.float64)
    z64 = (tc_z[0, 0] + jnp.sum(parts[:, 1, :])).astype(jnp.float64)
    n64 = (tc_n[0, 0] + jnp.sum(parts[:, 2, :])).astype(jnp.float64)
    t64 = (tc_t[0, 0] + jnp.sum(parts[:, 3, :])).astype(jnp.float64)
    return n64 * ROW_TLOGT - SVAL * t64 + SVAL * z64 + (SVAL - CONF) * g64


# speedup vs baseline: 2.6101x; 1.0298x over previous
import functools
import math

import numpy as np

import jax
import jax.numpy as jnp
from jax import lax
from jax.experimental import pallas as pl
from jax.experimental.pallas import tpu as pltpu
from jax.experimental.pallas import tpu_sc as plsc

jax.config.update("jax_enable_x64", True)

V = 32000
N = 2048
SMOOTHING = 0.1
CONF = 1.0 - SMOOTHING
SVAL = SMOOTHING / (V - 2)
ROW_TLOGT = (V - 2) * SVAL * math.log(SVAL) + CONF * math.log(CONF)

NTC = 1280
ROW_BLK = 256
COL_BLK = 3200

_NW = 32
_RPT = (N - NTC) // _NW
_CCH = 6400
_NCH = V // _CCH
_RPAD = ((_RPT + 15) // 16) * 16


def _tc_body(t_ref, x1_ref, x2_ref, tsum_ref, g_ref, z_ref, n_ref):
    i = pl.program_id(0)
    j = pl.program_id(1)

    @pl.when((i == 0) & (j == 0))
    def _init():
        tsum_ref[...] = jnp.zeros_like(tsum_ref)
        g_ref[...] = jnp.zeros_like(g_ref)
        z_ref[...] = jnp.zeros_like(z_ref)
        n_ref[...] = jnp.zeros_like(n_ref)

    w = (t_ref[...] != 0).astype(jnp.float32)
    xw1 = x1_ref[...] * w
    xw2 = x2_ref[...] * w
    tsum_ref[...] += jnp.sum(xw1 + xw2)
    cols = jax.lax.broadcasted_iota(jnp.int32, (ROW_BLK, COL_BLK), 1) \
        + j * COL_BLK
    t_blk = t_ref[...]
    g_ref[...] += jnp.sum(jnp.where(cols == t_blk, xw1, 0.0)
                          + jnp.where((cols + V // 2) == t_blk, xw2, 0.0))

    @pl.when(j == 0)
    def _col0():
        z_ref[...] += jnp.sum(xw1[:, 0:1])
        n_ref[...] += jnp.sum(w)


def _tc_sums(output, t2d):
    zero2 = lambda i, j: (jnp.int32(0), jnp.int32(0))
    one = jax.ShapeDtypeStruct((1, 1), jnp.float32)
    return pl.pallas_call(
        _tc_body,
        grid=(NTC // ROW_BLK, (V // 2) // COL_BLK),
        in_specs=[
            pl.BlockSpec((ROW_BLK, 1), lambda i, j: (i, jnp.int32(0))),
            pl.BlockSpec((ROW_BLK, COL_BLK), lambda i, j: (i, j)),
            pl.BlockSpec((ROW_BLK, COL_BLK),
                         lambda i, j: (i, j + (V // 2) // COL_BLK)),
        ],
        out_specs=[pl.BlockSpec((1, 1), zero2)] * 4,
        out_shape=[one, one, one, one],
        compiler_params=pltpu.CompilerParams(
            dimension_semantics=("arbitrary", "arbitrary")),
    )(t2d, output, output)


def _lane_splat(vec16, lane):
    return lax.gather(
        vec16, jnp.full((16, 1), lane, jnp.int32),
        lax.GatherDimensionNumbers(offset_dims=(),
                                   collapsed_slice_dims=(0,),
                                   start_index_map=(0,)),
        (1,), mode=lax.GatherScatterMode.PROMISE_IN_BOUNDS)


def _sc_partials(x2d, tgtf):
    mesh = plsc.VectorSubcoreMesh(core_axis_name="c", subcore_axis_name="s")

    @functools.partial(
        pl.kernel,
        mesh=mesh,
        out_type=jax.ShapeDtypeStruct((_NW, 4, 16), jnp.float32),
        scratch_types=[
            pltpu.VMEM((_RPAD,), jnp.float32),
            pltpu.VMEM((_RPAD,), jnp.float32),
            pltpu.VMEM((8, _CCH), jnp.float32),
            pltpu.VMEM((8, _CCH), jnp.float32),
            pltpu.VMEM((4, 16), jnp.float32),
            pltpu.SemaphoreType.DMA,
            pltpu.SemaphoreType.DMA,
        ],
    )
    def sc_k(x_hbm, tf_hbm, o_hbm, tf_v, wf_v, b0, b1, acc_v,
             sd0, sd1):
        wid = lax.axis_index("s") * 2 + lax.axis_index("c")
        rbase = NTC + wid * _RPT
        zeros = jnp.zeros((16,), jnp.float32)
        ones = jnp.ones((16,), jnp.float32)
        iota_f = lax.iota(jnp.int32, 16).astype(jnp.float32)
        e0 = jnp.maximum(ones - iota_f * iota_f, 0.0)
        for k in range(_RPAD // 16):
            tf_v[pl.ds(k * 16, 16)] = zeros
        pltpu.sync_copy(tf_hbm.at[pl.ds(rbase, _RPT)], tf_v.at[pl.ds(0, _RPT)])

        accn = zeros
        for k in range(_RPAD // 16):
            t16 = tf_v[pl.ds(k * 16, 16)]
            w16 = jnp.where(t16 != zeros, ones, zeros)
            if (k + 1) * 16 > _RPT:
                lanes = lax.iota(jnp.int32, 16)
                w16 = jnp.where(lanes < (_RPT - k * 16), w16, zeros)
            wf_v[pl.ds(k * 16, 16)] = w16
            accn = accn + w16

        seq = [(g, c) for g in range(_RPT // 8) for c in range(_NCH)]
        bufs = (b0, b1)
        sems = (sd0, sd1)

        def start(k):
            g, c = seq[k]
            return pltpu.async_copy(
                x_hbm.at[pl.ds(rbase + 8 * g, 8), pl.ds(c * _CCH, _CCH)],
                bufs[k % 2], sems[k % 2])

        handles = {0: start(0)}
        acct = zeros
        accg = zeros
        accz = zeros
        for k, (g, c) in enumerate(seq):
            if k + 1 < len(seq):
                handles[k + 1] = start(k + 1)
            handles[k].wait()
            buf = bufs[k % 2]
            for l in range(8):
                rlane = g * 8 + l
                tch = tf_v[pl.ds((rlane // 16) * 16, 16)]
                wch = wf_v[pl.ds((rlane // 16) * 16, 16)]
                tspl = _lane_splat(tch, rlane % 16)
                wspl = _lane_splat(wch, rlane % 16)
                cvec0 = iota_f + jnp.float32(c * _CCH)

                def inner(j, carry, buf=buf, l=l, tspl=tspl):
                    racc, gacc, cvec = carry
                    o = j * jnp.int32(64)
                    for u in range(4):
                        x16 = buf[l, pl.ds(o + jnp.int32(u * 16), 16)]
                        racc = racc + x16
                        gacc = gacc + jnp.where(cvec == tspl, x16, zeros)
                        cvec = cvec + jnp.float32(16.0)
                    return racc, gacc, cvec

                racc, gacc, _ = lax.fori_loop(
                    jnp.int32(0), jnp.int32(_CCH // 64), inner,
                    (zeros, zeros, cvec0))
                acct = acct + racc * wspl
                accg = accg + gacc * wspl
                if c == 0:
                    accz = accz + buf[l, pl.ds(0, 16)] * e0 * wspl
        acc_v[0] = accg
        acc_v[1] = accz
        acc_v[2] = accn
        acc_v[3] = acct
        pltpu.sync_copy(acc_v, o_hbm.at[wid])

    return sc_k(x2d, tgtf)


def kernel(output, target):
    tgt32 = target.astype(jnp.int32)
    tcol = tgt32[:, None]
    tc_t, tc_g, tc_z, tc_n = _tc_sums(output, tcol)
    parts = _sc_partials(output, tgt32.astype(jnp.float32))
    g64 = (tc_g[0, 0] + jnp.sum(parts[:, 0, :])).astype(jnp.float64)
    z64 = (tc_z[0, 0] + jnp.sum(parts[:, 1, :])).astype(jnp.float64)
    n64 = (tc_n[0, 0] + jnp.sum(parts[:, 2, :])).astype(jnp.float64)
    t64 = (tc_t[0, 0] + jnp.sum(parts[:, 3, :])).astype(jnp.float64)
    return n64 * ROW_TLOGT - SVAL * t64 + SVAL * z64 + (SVAL - CONF) * g64


# speedup vs baseline: 2.6962x; 1.0330x over previous
import functools
import math

import numpy as np

import jax
import jax.numpy as jnp
from jax import lax
from jax.experimental import pallas as pl
from jax.experimental.pallas import tpu as pltpu
from jax.experimental.pallas import tpu_sc as plsc

jax.config.update("jax_enable_x64", True)

V = 32000
N = 2048
SMOOTHING = 0.1
CONF = 1.0 - SMOOTHING
SVAL = SMOOTHING / (V - 2)
ROW_TLOGT = (V - 2) * SVAL * math.log(SVAL) + CONF * math.log(CONF)

NTC = 1280
ROW_BLK = 256
COL_BLK = 3200

_NW = 32
_RPT = (N - NTC) // _NW
_CCH = 6400
_NCH = V // _CCH
_RPAD = ((_RPT + 15) // 16) * 16


def _tc_body(t_ref, x1_ref, x2_ref, x3_ref, x4_ref, x5_ref,
             tsum_ref, g_ref, z_ref, n_ref):
    i = pl.program_id(0)
    j = pl.program_id(1)

    @pl.when((i == 0) & (j == 0))
    def _init():
        tsum_ref[...] = jnp.zeros_like(tsum_ref)
        g_ref[...] = jnp.zeros_like(g_ref)
        z_ref[...] = jnp.zeros_like(z_ref)
        n_ref[...] = jnp.zeros_like(n_ref)

    w = (t_ref[...] != 0).astype(jnp.float32)
    xws = [x * w for x in (x1_ref[...], x2_ref[...], x3_ref[...],
                           x4_ref[...], x5_ref[...])]
    tsum_ref[...] += jnp.sum(xws[0] + xws[1] + xws[2] + xws[3] + xws[4])
    cols = jax.lax.broadcasted_iota(jnp.int32, (ROW_BLK, COL_BLK), 1) \
        + j * COL_BLK
    t_blk = t_ref[...]
    gsum = jnp.where(cols == t_blk, xws[0], 0.0)
    for q in range(1, 5):
        gsum = gsum + jnp.where((cols + q * (V // 5)) == t_blk, xws[q], 0.0)
    g_ref[...] += jnp.sum(gsum)

    @pl.when(j == 0)
    def _col0():
        z_ref[...] += jnp.sum(xws[0][:, 0:1])
        n_ref[...] += jnp.sum(w)


def _tc_sums(output, t2d):
    zero2 = lambda i, j: (jnp.int32(0), jnp.int32(0))
    one = jax.ShapeDtypeStruct((1, 1), jnp.float32)
    return pl.pallas_call(
        _tc_body,
        grid=(NTC // ROW_BLK, (V // 5) // COL_BLK),
        in_specs=[pl.BlockSpec((ROW_BLK, 1), lambda i, j: (i, jnp.int32(0)))]
        + [pl.BlockSpec((ROW_BLK, COL_BLK),
                        functools.partial(
                            lambda q, i, j: (i, j + q * (V // 5) // COL_BLK),
                            q))
           for q in range(5)],
        out_specs=[pl.BlockSpec((1, 1), zero2)] * 4,
        out_shape=[one, one, one, one],
        compiler_params=pltpu.CompilerParams(
            dimension_semantics=("arbitrary", "arbitrary")),
    )(t2d, output, output, output, output, output)


def _lane_splat(vec16, lane):
    return lax.gather(
        vec16, jnp.full((16, 1), lane, jnp.int32),
        lax.GatherDimensionNumbers(offset_dims=(),
                                   collapsed_slice_dims=(0,),
                                   start_index_map=(0,)),
        (1,), mode=lax.GatherScatterMode.PROMISE_IN_BOUNDS)


def _sc_partials(x2d, tgtf):
    mesh = plsc.VectorSubcoreMesh(core_axis_name="c", subcore_axis_name="s")

    @functools.partial(
        pl.kernel,
        mesh=mesh,
        out_type=jax.ShapeDtypeStruct((_NW, 4, 16), jnp.float32),
        scratch_types=[
            pltpu.VMEM((_RPAD,), jnp.float32),
            pltpu.VMEM((_RPAD,), jnp.float32),
            pltpu.VMEM((8, _CCH), jnp.float32),
            pltpu.VMEM((8, _CCH), jnp.float32),
            pltpu.VMEM((4, 16), jnp.float32),
            pltpu.SemaphoreType.DMA,
            pltpu.SemaphoreType.DMA,
        ],
    )
    def sc_k(x_hbm, tf_hbm, o_hbm, tf_v, wf_v, b0, b1, acc_v,
             sd0, sd1):
        wid = lax.axis_index("s") * 2 + lax.axis_index("c")
        rbase = NTC + wid * _RPT
        zeros = jnp.zeros((16,), jnp.float32)
        ones = jnp.ones((16,), jnp.float32)
        iota_f = lax.iota(jnp.int32, 16).astype(jnp.float32)
        e0 = jnp.maximum(ones - iota_f * iota_f, 0.0)
        for k in range(_RPAD // 16):
            tf_v[pl.ds(k * 16, 16)] = zeros
        pltpu.sync_copy(tf_hbm.at[pl.ds(rbase, _RPT)], tf_v.at[pl.ds(0, _RPT)])

        accn = zeros
        for k in range(_RPAD // 16):
            t16 = tf_v[pl.ds(k * 16, 16)]
            w16 = jnp.where(t16 != zeros, ones, zeros)
            if (k + 1) * 16 > _RPT:
                lanes = lax.iota(jnp.int32, 16)
                w16 = jnp.where(lanes < (_RPT - k * 16), w16, zeros)
            wf_v[pl.ds(k * 16, 16)] = w16
            accn = accn + w16

        seq = [(g, c) for g in range(_RPT // 8) for c in range(_NCH)]
        bufs = (b0, b1)
        sems = (sd0, sd1)

        def start(k):
            g, c = seq[k]
            return pltpu.async_copy(
                x_hbm.at[pl.ds(rbase + 8 * g, 8), pl.ds(c * _CCH, _CCH)],
                bufs[k % 2], sems[k % 2])

        handles = {0: start(0)}
        acct = zeros
        accg = zeros
        accz = zeros
        for k, (g, c) in enumerate(seq):
            if k + 1 < len(seq):
                handles[k + 1] = start(k + 1)
            handles[k].wait()
            buf = bufs[k % 2]
            for l in range(8):
                rlane = g * 8 + l
                tch = tf_v[pl.ds((rlane // 16) * 16, 16)]
                wch = wf_v[pl.ds((rlane // 16) * 16, 16)]
                tspl = _lane_splat(tch, rlane % 16)
                wspl = _lane_splat(wch, rlane % 16)
                cvec0 = iota_f + jnp.float32(c * _CCH)

                def inner(j, carry, buf=buf, l=l, tspl=tspl):
                    racc, gacc, cvec = carry
                    o = j * jnp.int32(64)
                    for u in range(4):
                        x16 = buf[l, pl.ds(o + jnp.int32(u * 16), 16)]
                        racc = racc + x16
                        gacc = gacc + jnp.where(cvec == tspl, x16, zeros)
                        cvec = cvec + jnp.float32(16.0)
                    return racc, gacc, cvec

                racc, gacc, _ = lax.fori_loop(
                    jnp.int32(0), jnp.int32(_CCH // 64), inner,
                    (zeros, zeros, cvec0))
                acct = acct + racc * wspl
                accg = accg + gacc * wspl
                if c == 0:
                    accz = accz + buf[l, pl.ds(0, 16)] * e0 * wspl
        acc_v[0] = accg
        acc_v[1] = accz
        acc_v[2] = accn
        acc_v[3] = acct
        pltpu.sync_copy(acc_v, o_hbm.at[wid])

    return sc_k(x2d, tgtf)


def kernel(output, target):
    tgt32 = target.astype(jnp.int32)
    tcol = tgt32[:, None]
    tc_t, tc_g, tc_z, tc_n = _tc_sums(output, tcol)
    parts = _sc_partials(output, tgt32.astype(jnp.float32))
    g64 = (tc_g[0, 0] + jnp.sum(parts[:, 0, :])).astype(jnp.float64)
    z64 = (tc_z[0, 0] + jnp.sum(parts[:, 1, :])).astype(jnp.float64)
    n64 = (tc_n[0, 0] + jnp.sum(parts[:, 2, :])).astype(jnp.float64)
    t64 = (tc_t[0, 0] + jnp.sum(parts[:, 3, :])).astype(jnp.float64)
    return n64 * ROW_TLOGT - SVAL * t64 + SVAL * z64 + (SVAL - CONF) * g64


# speedup vs baseline: 2.6970x; 1.0003x over previous
import functools
import math

import numpy as np

import jax
import jax.numpy as jnp
from jax import lax
from jax.experimental import pallas as pl
from jax.experimental.pallas import tpu as pltpu
from jax.experimental.pallas import tpu_sc as plsc

jax.config.update("jax_enable_x64", True)

V = 32000
N = 2048
SMOOTHING = 0.1
CONF = 1.0 - SMOOTHING
SVAL = SMOOTHING / (V - 2)
ROW_TLOGT = (V - 2) * SVAL * math.log(SVAL) + CONF * math.log(CONF)

NTC = 1280
ROW_BLK = 256
COL_BLK = 3200

_NW = 32
_RPT = (N - NTC) // _NW
_CCH = 6400
_NCH = V // _CCH
_RPAD = ((_RPT + 15) // 16) * 16


def _tc_body(t_ref, x1_ref, x2_ref, x3_ref, x4_ref, x5_ref,
             tsum_ref, g_ref, z_ref, n_ref):
    i = pl.program_id(0)
    j = pl.program_id(1)

    @pl.when((i == 0) & (j == 0))
    def _init():
        tsum_ref[...] = jnp.zeros_like(tsum_ref)
        g_ref[...] = jnp.zeros_like(g_ref)
        z_ref[...] = jnp.zeros_like(z_ref)
        n_ref[...] = jnp.zeros_like(n_ref)

    w = (t_ref[...] != 0).astype(jnp.float32)
    xws = [x * w for x in (x1_ref[...], x2_ref[...], x3_ref[...],
                           x4_ref[...], x5_ref[...])]
    tsum_ref[...] += jnp.sum(xws[0] + xws[1] + xws[2] + xws[3] + xws[4])
    cols = jax.lax.broadcasted_iota(jnp.int32, (ROW_BLK, COL_BLK), 1) \
        + j * COL_BLK
    t_blk = t_ref[...]
    gsum = jnp.where(cols == t_blk, xws[0], 0.0)
    for q in range(1, 5):
        gsum = gsum + jnp.where((cols + q * (V // 5)) == t_blk, xws[q], 0.0)
    g_ref[...] += jnp.sum(gsum)

    @pl.when(j == 0)
    def _col0():
        z_ref[...] += jnp.sum(xws[0][:, 0:1])
        n_ref[...] += jnp.sum(w)


def _tc_sums(output, t2d):
    zero2 = lambda i, j: (jnp.int32(0), jnp.int32(0))
    one = jax.ShapeDtypeStruct((1, 1), jnp.float32)
    return pl.pallas_call(
        _tc_body,
        grid=(NTC // ROW_BLK, (V // 5) // COL_BLK),
        in_specs=[pl.BlockSpec((ROW_BLK, 1), lambda i, j: (i, jnp.int32(0)))]
        + [pl.BlockSpec((ROW_BLK, COL_BLK),
                        functools.partial(
                            lambda q, i, j: (i, j + q * (V // 5) // COL_BLK),
                            q))
           for q in range(5)],
        out_specs=[pl.BlockSpec((1, 1), zero2)] * 4,
        out_shape=[one, one, one, one],
        compiler_params=pltpu.CompilerParams(
            dimension_semantics=("arbitrary", "arbitrary")),
    )(t2d, output, output, output, output, output)


def _lane_splat(vec16, lane):
    return lax.gather(
        vec16, jnp.full((16, 1), lane, jnp.int32),
        lax.GatherDimensionNumbers(offset_dims=(),
                                   collapsed_slice_dims=(0,),
                                   start_index_map=(0,)),
        (1,), mode=lax.GatherScatterMode.PROMISE_IN_BOUNDS)


def _sc_partials(x2d, tgtf):
    mesh = plsc.VectorSubcoreMesh(core_axis_name="c", subcore_axis_name="s")

    @functools.partial(
        pl.kernel,
        mesh=mesh,
        out_type=jax.ShapeDtypeStruct((_NW, 4, 16), jnp.float32),
        scratch_types=[
            pltpu.VMEM((_RPAD,), jnp.float32),
            pltpu.VMEM((_RPAD,), jnp.float32),
            pltpu.VMEM((8, _CCH), jnp.float32),
            pltpu.VMEM((8, _CCH), jnp.float32),
            pltpu.VMEM((4, 16), jnp.float32),
            pltpu.SemaphoreType.DMA,
            pltpu.SemaphoreType.DMA,
        ],
    )
    def sc_k(x_hbm, tf_hbm, o_hbm, tf_v, wf_v, b0, b1, acc_v,
             sd0, sd1):
        wid = lax.axis_index("s") * 2 + lax.axis_index("c")
        rbase = NTC + wid * _RPT
        zeros = jnp.zeros((16,), jnp.float32)
        ones = jnp.ones((16,), jnp.float32)
        iota_f = lax.iota(jnp.int32, 16).astype(jnp.float32)
        e0 = jnp.maximum(ones - iota_f * iota_f, 0.0)
        for k in range(_RPAD // 16):
            tf_v[pl.ds(k * 16, 16)] = zeros
        pltpu.sync_copy(tf_hbm.at[pl.ds(rbase, _RPT)], tf_v.at[pl.ds(0, _RPT)])

        accn = zeros
        for k in range(_RPAD // 16):
            t16 = tf_v[pl.ds(k * 16, 16)]
            w16 = jnp.where(t16 != zeros, ones, zeros)
            if (k + 1) * 16 > _RPT:
                lanes = lax.iota(jnp.int32, 16)
                w16 = jnp.where(lanes < (_RPT - k * 16), w16, zeros)
            wf_v[pl.ds(k * 16, 16)] = w16
            accn = accn + w16

        seq = [(g, c) for g in range(_RPT // 8) for c in range(_NCH)]
        bufs = (b0, b1)
        sems = (sd0, sd1)

        def start(k):
            g, c = seq[k]
            return pltpu.async_copy(
                x_hbm.at[pl.ds(rbase + 8 * g, 8), pl.ds(c * _CCH, _CCH)],
                bufs[k % 2], sems[k % 2])

        handles = {0: start(0)}
        acct = zeros
        accg = zeros
        accz = zeros
        for k, (g, c) in enumerate(seq):
            if k + 1 < len(seq):
                handles[k + 1] = start(k + 1)
            handles[k].wait()
            buf = bufs[k % 2]
            for l in range(8):
                rlane = g * 8 + l
                tch = tf_v[pl.ds((rlane // 16) * 16, 16)]
                wch = wf_v[pl.ds((rlane // 16) * 16, 16)]
                tspl = _lane_splat(tch, rlane % 16)
                wspl = _lane_splat(wch, rlane % 16)
                cvec0 = iota_f + jnp.float32(c * _CCH)

                def inner(j, carry, buf=buf, l=l, tspl=tspl):
                    racc, gacc, cvec = carry
                    o = j * jnp.int32(64)
                    for u in range(4):
                        x16 = buf[l, pl.ds(o + jnp.int32(u * 16), 16)]
                        racc = racc + x16
                        gacc = gacc + jnp.where(cvec == tspl, x16, zeros)
                        cvec = cvec + jnp.float32(16.0)
                    return racc, gacc, cvec

                racc, gacc, _ = lax.fori_loop(
                    jnp.int32(0), jnp.int32(_CCH // 64), inner,
                    (zeros, zeros, cvec0))
                acct = acct + racc * wspl
                accg = accg + gacc * wspl
                if c == 0:
                    accz = accz + buf[l, pl.ds(0, 16)] * e0 * wspl
        acc_v[0] = accg
        acc_v[1] = accz
        acc_v[2] = accn
        acc_v[3] = acct
        pltpu.sync_copy(acc_v, o_hbm.at[wid])

    return sc_k(x2d, tgtf)


def kernel(output, target):
    tgt32 = target.astype(jnp.int32)
    tcol = tgt32[:, None]
    tc_t, tc_g, tc_z, tc_n = _tc_sums(output, tcol)
    parts = _sc_partials(output, tgt32.astype(jnp.float32))
    g = tc_g[0, 0] + jnp.sum(parts[:, 0, :])
    z = tc_z[0, 0] + jnp.sum(parts[:, 1, :])
    n = tc_n[0, 0] + jnp.sum(parts[:, 2, :])
    t = tc_t[0, 0] + jnp.sum(parts[:, 3, :])
    loss32 = (n * jnp.float32(ROW_TLOGT) - jnp.float32(SVAL) * t
              + jnp.float32(SVAL) * z + jnp.float32(SVAL - CONF) * g)
    return loss32.astype(jnp.float64)


# speedup vs baseline: 2.6979x; 1.0003x over previous
import functools
import math

import jax
import jax.numpy as jnp
from jax import lax
from jax.experimental import pallas as pl
from jax.experimental.pallas import tpu as pltpu
from jax.experimental.pallas import tpu_sc as plsc

jax.config.update("jax_enable_x64", True)

V = 32000
N = 2048
SMOOTHING = 0.1
CONF = 1.0 - SMOOTHING
SVAL = SMOOTHING / (V - 2)
ROW_TLOGT = (V - 2) * SVAL * math.log(SVAL) + CONF * math.log(CONF)

NTC = 1280
ROW_BLK = 256
COL_BLK = 3200

_NW = 32
_RPT = (N - NTC) // _NW
_CCH = 6400
_NCH = V // _CCH
_RPAD = ((_RPT + 15) // 16) * 16


def _tc_body(t_ref, x1_ref, x2_ref, x3_ref, x4_ref, x5_ref,
             tsum_ref, g_ref, z_ref, n_ref):
    i = pl.program_id(0)
    j = pl.program_id(1)

    @pl.when((i == 0) & (j == 0))
    def _init():
        tsum_ref[...] = jnp.zeros_like(tsum_ref)
        g_ref[...] = jnp.zeros_like(g_ref)
        z_ref[...] = jnp.zeros_like(z_ref)
        n_ref[...] = jnp.zeros_like(n_ref)

    w = (t_ref[...] != 0).astype(jnp.float32)
    xws = [x * w for x in (x1_ref[...], x2_ref[...], x3_ref[...],
                           x4_ref[...], x5_ref[...])]
    tsum_ref[...] += jnp.sum(xws[0] + xws[1] + xws[2] + xws[3] + xws[4])
    cols = jax.lax.broadcasted_iota(jnp.int32, (ROW_BLK, COL_BLK), 1) \
        + j * COL_BLK
    t_blk = t_ref[...]
    gsum = jnp.where(cols == t_blk, xws[0], 0.0)
    for q in range(1, 5):
        gsum = gsum + jnp.where((cols + q * (V // 5)) == t_blk, xws[q], 0.0)
    g_ref[...] += jnp.sum(gsum)

    @pl.when(j == 0)
    def _col0():
        z_ref[...] += jnp.sum(xws[0][:, 0:1])
        n_ref[...] += jnp.sum(w)


def _tc_sums(output, t2d):
    zero2 = lambda i, j: (jnp.int32(0), jnp.int32(0))
    one = jax.ShapeDtypeStruct((1, 1), jnp.float32)
    return pl.pallas_call(
        _tc_body,
        grid=(NTC // ROW_BLK, (V // 5) // COL_BLK),
        in_specs=[pl.BlockSpec((ROW_BLK, 1), lambda i, j: (i, jnp.int32(0)))]
        + [pl.BlockSpec((ROW_BLK, COL_BLK),
                        functools.partial(
                            lambda q, i, j: (i, j + q * (V // 5) // COL_BLK),
                            q))
           for q in range(5)],
        out_specs=[pl.BlockSpec((1, 1), zero2)] * 4,
        out_shape=[one, one, one, one],
        compiler_params=pltpu.CompilerParams(
            dimension_semantics=("arbitrary", "arbitrary")),
    )(t2d, output, output, output, output, output)


def _lane_splat(vec16, lane):
    return lax.gather(
        vec16, jnp.full((16, 1), lane, jnp.int32),
        lax.GatherDimensionNumbers(offset_dims=(),
                                   collapsed_slice_dims=(0,),
                                   start_index_map=(0,)),
        (1,), mode=lax.GatherScatterMode.PROMISE_IN_BOUNDS)


def _sc_partials(x2d, tgtf):
    mesh = plsc.VectorSubcoreMesh(core_axis_name="c", subcore_axis_name="s")

    @functools.partial(
        pl.kernel,
        mesh=mesh,
        out_type=jax.ShapeDtypeStruct((_NW, 4, 16), jnp.float32),
        scratch_types=[
            pltpu.VMEM((_RPAD,), jnp.float32),
            pltpu.VMEM((_RPAD,), jnp.float32),
            pltpu.VMEM((8, _CCH), jnp.float32),
            pltpu.VMEM((8, _CCH), jnp.float32),
            pltpu.VMEM((4, 16), jnp.float32),
            pltpu.SemaphoreType.DMA,
            pltpu.SemaphoreType.DMA,
        ],
    )
    def sc_k(x_hbm, tf_hbm, o_hbm, tf_v, wf_v, b0, b1, acc_v,
             sd0, sd1):
        wid = lax.axis_index("s") * 2 + lax.axis_index("c")
        rbase = NTC + wid * _RPT
        zeros = jnp.zeros((16,), jnp.float32)
        ones = jnp.ones((16,), jnp.float32)
        iota_f = lax.iota(jnp.int32, 16).astype(jnp.float32)
        e0 = jnp.maximum(ones - iota_f * iota_f, 0.0)
        for k in range(_RPAD // 16):
            tf_v[pl.ds(k * 16, 16)] = zeros
        pltpu.sync_copy(tf_hbm.at[pl.ds(rbase, _RPT)], tf_v.at[pl.ds(0, _RPT)])

        accn = zeros
        for k in range(_RPAD // 16):
            t16 = tf_v[pl.ds(k * 16, 16)]
            w16 = jnp.where(t16 != zeros, ones, zeros)
            if (k + 1) * 16 > _RPT:
                lanes = lax.iota(jnp.int32, 16)
                w16 = jnp.where(lanes < (_RPT - k * 16), w16, zeros)
            wf_v[pl.ds(k * 16, 16)] = w16
            accn = accn + w16

        seq = [(g, c) for g in range(_RPT // 8) for c in range(_NCH)]
        bufs = (b0, b1)
        sems = (sd0, sd1)

        def start(k):
            g, c = seq[k]
            return pltpu.async_copy(
                x_hbm.at[pl.ds(rbase + 8 * g, 8), pl.ds(c * _CCH, _CCH)],
                bufs[k % 2], sems[k % 2])

        handles = {0: start(0)}
        acct = zeros
        accg = zeros
        accz = zeros
        for k, (g, c) in enumerate(seq):
            if k + 1 < len(seq):
                handles[k + 1] = start(k + 1)
            handles[k].wait()
            buf = bufs[k % 2]
            for l in range(8):
                rlane = g * 8 + l
                tch = tf_v[pl.ds((rlane // 16) * 16, 16)]
                wch = wf_v[pl.ds((rlane // 16) * 16, 16)]
                tspl = _lane_splat(tch, rlane % 16)
                wspl = _lane_splat(wch, rlane % 16)
                cvec0 = iota_f + jnp.float32(c * _CCH)

                def inner(j, carry, buf=buf, l=l, tspl=tspl):
                    racc, gacc, cvec = carry
                    o = j * jnp.int32(64)
                    for u in range(4):
                        x16 = buf[l, pl.ds(o + jnp.int32(u * 16), 16)]
                        racc = racc + x16
                        gacc = gacc + jnp.where(cvec == tspl, x16, zeros)
                        cvec = cvec + jnp.float32(16.0)
                    return racc, gacc, cvec

                racc, gacc, _ = lax.fori_loop(
                    jnp.int32(0), jnp.int32(_CCH // 64), inner,
                    (zeros, zeros, cvec0))
                acct = acct + racc * wspl
                accg = accg + gacc * wspl
                if c == 0:
                    accz = accz + buf[l, pl.ds(0, 16)] * e0 * wspl
        acc_v[0] = accg
        acc_v[1] = accz
        acc_v[2] = accn
        acc_v[3] = acct
        pltpu.sync_copy(acc_v, o_hbm.at[wid])

    return sc_k(x2d, tgtf)


def kernel(output, target):
    tgt32 = target.astype(jnp.int32)
    tcol = tgt32[:, None]
    tc_t, tc_g, tc_z, tc_n = _tc_sums(output, tcol)
    parts = _sc_partials(output, tgt32.astype(jnp.float32))
    g = tc_g[0, 0] + jnp.sum(parts[:, 0, :])
    z = tc_z[0, 0] + jnp.sum(parts[:, 1, :])
    n = tc_n[0, 0] + jnp.sum(parts[:, 2, :])
    t = tc_t[0, 0] + jnp.sum(parts[:, 3, :])
    loss32 = (n * jnp.float32(ROW_TLOGT) - jnp.float32(SVAL) * t
              + jnp.float32(SVAL) * z + jnp.float32(SVAL - CONF) * g)
    return loss32.astype(jnp.float64)
